# baseline (speedup 1.0000x reference)
import functools

import jax
import jax.numpy as jnp
from jax import lax
from jax.experimental import pallas as pl
from jax.experimental.pallas import tpu as pltpu
from jax.experimental.pallas import tpu_sc as plsc

B = 16384
D = 32
L = 16


def _recommender_sc(user_ids, movie_ids, user_emb, movie_emb,
                    user_bias1d, movie_bias1d):
    info = plsc.get_sparse_core_info()
    nc, ns = info.num_cores, info.num_subcores
    nw = nc * ns
    bpw = B // nw

    mesh = plsc.VectorSubcoreMesh(core_axis_name="c", subcore_axis_name="s")

    @functools.partial(
        pl.kernel,
        mesh=mesh,
        compiler_params=pltpu.CompilerParams(
            needs_layout_passes=False, use_tc_tiling_on_sc=False),
        out_type=jax.ShapeDtypeStruct((B,), jnp.float32),
        scratch_types=[
            pltpu.VMEM((bpw,), jnp.int32),
            pltpu.VMEM((bpw,), jnp.int32),
            pltpu.VMEM((bpw, D), jnp.float32),
            pltpu.VMEM((bpw, D), jnp.float32),
            pltpu.VMEM((bpw,), jnp.float32),
            pltpu.VMEM((bpw,), jnp.float32),
            pltpu.VMEM((bpw,), jnp.float32),
            pltpu.SemaphoreType.DMA,
        ],
    )
    def k(uids_hbm, mids_hbm, uemb_hbm, memb_hbm, ub_hbm, mb_hbm, out_hbm,
          uidx, midx, urows, mrows, ubias, mbias, outv, sem):
        wid = lax.axis_index("s") * nc + lax.axis_index("c")
        base = wid * bpw

        pltpu.sync_copy(uids_hbm.at[pl.ds(base, bpw)], uidx)
        pltpu.sync_copy(mids_hbm.at[pl.ds(base, bpw)], midx)

        c1 = pltpu.async_copy(uemb_hbm.at[uidx], urows, sem)
        c2 = pltpu.async_copy(memb_hbm.at[midx], mrows, sem)
        c3 = pltpu.async_copy(ub_hbm.at[uidx], ubias, sem)
        c4 = pltpu.async_copy(mb_hbm.at[midx], mbias, sem)
        c1.wait()
        c2.wait()
        c3.wait()
        c4.wait()

        lane = lax.iota(jnp.int32, L)

        def group(g, _):
            sl = pl.ds(g * L, L)
            acc = ubias[sl] + mbias[sl]
            for r in range(L):
                i = g * L + r
                s = (urows[i, pl.ds(0, L)] * mrows[i, pl.ds(0, L)]
                     + urows[i, pl.ds(L, L)] * mrows[i, pl.ds(L, L)])
                acc = acc + jnp.where(lane == r, jnp.sum(s), 0.0)
            outv[sl] = acc
            return 0

        lax.fori_loop(0, bpw // L, group, 0)

        pltpu.sync_copy(outv, out_hbm.at[pl.ds(base, bpw)])

    return k(user_ids, movie_ids, user_emb, movie_emb,
             user_bias1d, movie_bias1d)


def kernel(user_ids, movie_ids, user_emb, movie_emb, user_bias, movie_bias):
    return _recommender_sc(
        user_ids.astype(jnp.int32),
        movie_ids.astype(jnp.int32),
        user_emb,
        movie_emb,
        user_bias.reshape(-1),
        movie_bias.reshape(-1),
    )
